# Initial kernel scaffold; baseline (speedup 1.0000x reference)
import jax
import jax.numpy as jnp
from jax.experimental import pallas as pl


def kernel(x, embedding):
    raise NotImplementedError("write your pallas kernel here")


# speedup vs baseline: 1.8734x; 1.8734x over previous
import jax
import jax.numpy as jnp
from jax.experimental import pallas as pl
from jax.experimental.pallas import tpu as pltpu

_TSZ = 4096
_K = 8192
_D = 256
_GAMMA = 0.25

_BR = 1024
_BC = 2048


def _main_kernel(x_ref, e_ref, d_ref, m_ref, a_ref, runm_ref, runa_ref):
    j = pl.program_id(1)
    nc = pl.num_programs(1)

    ze = x_ref[...]
    an = jnp.sqrt(jnp.sum(ze * ze, axis=1, keepdims=True))
    ze_n = ze / jnp.maximum(an, 1e-8)

    e = e_ref[...]
    n1 = jnp.sqrt(jnp.sum(e * e, axis=1, keepdims=True))
    p1 = e / jnp.maximum(n1, 1e-12)
    n2 = jnp.sqrt(jnp.sum(p1 * p1, axis=1, keepdims=True))
    p2 = p1 / jnp.maximum(n2, 1e-8)

    dt = jax.lax.dot_general(
        ze_n, p2, (((1,), (1,)), ((), ())),
        preferred_element_type=jnp.float32,
    )
    d_ref[...] = dt

    tile_max = jnp.max(dt, axis=1, keepdims=True)
    iota = jax.lax.broadcasted_iota(jnp.int32, dt.shape, 1)
    masked = jnp.where(dt == tile_max, iota, _K)
    tile_arg = jnp.min(masked, axis=1, keepdims=True) + j * _BC

    @pl.when(j == 0)
    def _():
        runm_ref[...] = tile_max
        runa_ref[...] = tile_arg

    @pl.when(j > 0)
    def _():
        better = tile_max > runm_ref[...]
        runa_ref[...] = jnp.where(better, tile_arg, runa_ref[...])
        runm_ref[...] = jnp.maximum(tile_max, runm_ref[...])

    @pl.when(j == nc - 1)
    def _():
        m_ref[...] = runm_ref[...]
        a_ref[...] = runa_ref[...]


def _loss_kernel(m_ref, o_ref):
    m = m_ref[...]
    logp = jnp.maximum(jnp.log(jnp.clip(m, 1e-12, 1.0)), -100.0)
    o_ref[...] = jnp.reshape(-jnp.mean(logp) * (1.0 + _GAMMA), (1, 1))


def kernel(x, embedding):
    ze = x.reshape(_TSZ, _D)
    protos = embedding.reshape(_K, _D)
    d, m, a = pl.pallas_call(
        _main_kernel,
        grid=(_TSZ // _BR, _K // _BC),
        in_specs=[
            pl.BlockSpec((_BR, _D), lambda i, j: (i, 0)),
            pl.BlockSpec((_BC, _D), lambda i, j: (j, 0)),
        ],
        out_specs=[
            pl.BlockSpec((_BR, _BC), lambda i, j: (i, j)),
            pl.BlockSpec((_BR, 1), lambda i, j: (i, 0)),
            pl.BlockSpec((_BR, 1), lambda i, j: (i, 0)),
        ],
        out_shape=[
            jax.ShapeDtypeStruct((_TSZ, _K), jnp.float32),
            jax.ShapeDtypeStruct((_TSZ, 1), jnp.float32),
            jax.ShapeDtypeStruct((_TSZ, 1), jnp.int32),
        ],
        scratch_shapes=[
            pltpu.VMEM((_BR, 1), jnp.float32),
            pltpu.VMEM((_BR, 1), jnp.int32),
        ],
        compiler_params=pltpu.CompilerParams(
            dimension_semantics=("parallel", "arbitrary"),
        ),
    )(ze, protos)
    loss = pl.pallas_call(
        _loss_kernel,
        out_shape=jax.ShapeDtypeStruct((1, 1), jnp.float32),
    )(m.reshape(1, _TSZ))
    return d, a.reshape(_TSZ), loss.reshape(())


# speedup vs baseline: 2.0087x; 1.0722x over previous
import jax
import jax.numpy as jnp
from jax.experimental import pallas as pl
from jax.experimental.pallas import tpu as pltpu

_TSZ = 4096
_K = 8192
_D = 256
_GAMMA = 0.25

_BR = 1024
_BC = 2048
_NI = _TSZ // _BR
_NJ = _K // _BC


def _main_kernel(x_ref, e_ref, d_ref, a_ref, loss_ref,
                 zen_ref, en_ref, runm_ref, runa_ref, acc_ref):
    j = pl.program_id(0)
    i = pl.program_id(1)

    @pl.when(i == 0)
    def _():
        e = e_ref[...]
        n1 = jnp.sqrt(jnp.sum(e * e, axis=1, keepdims=True))
        p1 = e / jnp.maximum(n1, 1e-12)
        n2 = jnp.sqrt(jnp.sum(p1 * p1, axis=1, keepdims=True))
        en_ref[...] = p1 / jnp.maximum(n2, 1e-8)

    @pl.when(j == 0)
    def _():
        ze = x_ref[...]
        an = jnp.sqrt(jnp.sum(ze * ze, axis=1, keepdims=True))
        zen_ref[pl.ds(i * _BR, _BR), :] = ze / jnp.maximum(an, 1e-8)

    dt = jax.lax.dot_general(
        zen_ref[pl.ds(i * _BR, _BR), :], en_ref[...],
        (((1,), (1,)), ((), ())),
        preferred_element_type=jnp.float32,
    )
    d_ref[...] = dt

    tile_max = jnp.max(dt, axis=1, keepdims=True)
    iota = jax.lax.broadcasted_iota(jnp.int32, dt.shape, 1)
    masked = jnp.where(dt == tile_max, iota, _K)
    tile_arg = jnp.min(masked, axis=1, keepdims=True) + j * _BC

    rows = pl.ds(i * _BR, _BR)

    @pl.when(j == 0)
    def _():
        runm_ref[rows, :] = tile_max
        runa_ref[rows, :] = tile_arg

    @pl.when(j > 0)
    def _():
        better = tile_max > runm_ref[rows, :]
        runa_ref[rows, :] = jnp.where(better, tile_arg, runa_ref[rows, :])
        runm_ref[rows, :] = jnp.maximum(tile_max, runm_ref[rows, :])

    @pl.when(j == _NJ - 1)
    def _():
        m = runm_ref[rows, :]
        a_ref[...] = runa_ref[rows, :]
        logp = jnp.maximum(jnp.log(jnp.clip(m, 1e-12, 1.0)), -100.0)
        part = jnp.sum(logp) * (-(1.0 + _GAMMA) / _TSZ)

        @pl.when(i == 0)
        def _():
            acc_ref[0] = part

        @pl.when(i > 0)
        def _():
            acc_ref[0] = acc_ref[0] + part

        @pl.when(i == _NI - 1)
        def _():
            loss_ref[...] = jnp.reshape(acc_ref[0], (1, 1))


def kernel(x, embedding):
    ze = x.reshape(_TSZ, _D)
    protos = embedding.reshape(_K, _D)
    d, a, loss = pl.pallas_call(
        _main_kernel,
        grid=(_NJ, _NI),
        in_specs=[
            pl.BlockSpec((_BR, _D), lambda j, i: (i, 0)),
            pl.BlockSpec((_BC, _D), lambda j, i: (j, 0)),
        ],
        out_specs=[
            pl.BlockSpec((_BR, _BC), lambda j, i: (i, j)),
            pl.BlockSpec((_BR, 1), lambda j, i: (i, 0)),
            pl.BlockSpec((1, 1), lambda j, i: (0, 0)),
        ],
        out_shape=[
            jax.ShapeDtypeStruct((_TSZ, _K), jnp.float32),
            jax.ShapeDtypeStruct((_TSZ, 1), jnp.int32),
            jax.ShapeDtypeStruct((1, 1), jnp.float32),
        ],
        scratch_shapes=[
            pltpu.VMEM((_TSZ, _D), jnp.float32),
            pltpu.VMEM((_BC, _D), jnp.float32),
            pltpu.VMEM((_TSZ, 1), jnp.float32),
            pltpu.VMEM((_TSZ, 1), jnp.int32),
            pltpu.SMEM((1,), jnp.float32),
        ],
        compiler_params=pltpu.CompilerParams(
            dimension_semantics=("arbitrary", "arbitrary"),
        ),
    )(ze, protos)
    return d, a.reshape(_TSZ), loss.reshape(())


# speedup vs baseline: 2.3182x; 1.1541x over previous
import jax
import jax.numpy as jnp
from jax.experimental import pallas as pl
from jax.experimental.pallas import tpu as pltpu

_TSZ = 4096
_K = 8192
_D = 256
_GAMMA = 0.25

_BR = 1024
_BC = 2048
_NI = _TSZ // _BR
_NJ = _K // _BC


def _main_kernel(x_ref, e_ref, d_ref, a_ref, loss_ref,
                 zen_ref, en_ref, runm_ref, runa_ref, acc_ref):
    j = pl.program_id(0)
    i = pl.program_id(1)

    @pl.when(i == 0)
    def _():
        e = e_ref[:, 0, :]
        n1 = jnp.sqrt(jnp.sum(e * e, axis=1, keepdims=True))
        p1 = e / jnp.maximum(n1, 1e-12)
        n2 = jnp.sqrt(jnp.sum(p1 * p1, axis=1, keepdims=True))
        en_ref[...] = p1 / jnp.maximum(n2, 1e-8)

    @pl.when(j == 0)
    def _():
        ze = x_ref[0]
        an = jnp.sqrt(jnp.sum(ze * ze, axis=1, keepdims=True))
        zen_ref[pl.ds(i * _BR, _BR), :] = ze / jnp.maximum(an, 1e-8)

    dt = jax.lax.dot_general(
        zen_ref[pl.ds(i * _BR, _BR), :], en_ref[...],
        (((1,), (1,)), ((), ())),
        preferred_element_type=jnp.float32,
    )
    d_ref[...] = dt

    tile_max = jnp.max(dt, axis=1, keepdims=True)
    iota = jax.lax.broadcasted_iota(jnp.int32, dt.shape, 1)
    masked = jnp.where(dt == tile_max, iota, _K)
    tile_arg = jnp.min(masked, axis=1, keepdims=True) + j * _BC

    rows = pl.ds(i * _BR, _BR)

    @pl.when(j == 0)
    def _():
        runm_ref[rows, :] = tile_max
        runa_ref[rows, :] = tile_arg

    @pl.when(j > 0)
    def _():
        better = tile_max > runm_ref[rows, :]
        runa_ref[rows, :] = jnp.where(better, tile_arg, runa_ref[rows, :])
        runm_ref[rows, :] = jnp.maximum(tile_max, runm_ref[rows, :])

    @pl.when(j == _NJ - 1)
    def _():
        m = runm_ref[rows, :]
        a_ref[...] = runa_ref[rows, :]
        logp = jnp.maximum(jnp.log(jnp.clip(m, 1e-12, 1.0)), -100.0)
        part = jnp.sum(logp) * (-(1.0 + _GAMMA) / _TSZ)

        @pl.when(i == 0)
        def _():
            acc_ref[0] = part

        @pl.when(i > 0)
        def _():
            acc_ref[0] = acc_ref[0] + part

        @pl.when(i == _NI - 1)
        def _():
            loss_ref[...] = jnp.reshape(acc_ref[0], (1, 1))


def kernel(x, embedding):
    d, a, loss = pl.pallas_call(
        _main_kernel,
        grid=(_NJ, _NI),
        in_specs=[
            pl.BlockSpec((1, _BR, _D), lambda j, i: (0, i, 0)),
            pl.BlockSpec((_BC, 1, _D), lambda j, i: (j, 0, 0)),
        ],
        out_specs=[
            pl.BlockSpec((_BR, _BC), lambda j, i: (i, j)),
            pl.BlockSpec((_BR, 1), lambda j, i: (i, 0)),
            pl.BlockSpec((1, 1), lambda j, i: (0, 0)),
        ],
        out_shape=[
            jax.ShapeDtypeStruct((_TSZ, _K), jnp.float32),
            jax.ShapeDtypeStruct((_TSZ, 1), jnp.int32),
            jax.ShapeDtypeStruct((1, 1), jnp.float32),
        ],
        scratch_shapes=[
            pltpu.VMEM((_TSZ, _D), jnp.float32),
            pltpu.VMEM((_BC, _D), jnp.float32),
            pltpu.VMEM((_TSZ, 1), jnp.float32),
            pltpu.VMEM((_TSZ, 1), jnp.int32),
            pltpu.SMEM((1,), jnp.float32),
        ],
        compiler_params=pltpu.CompilerParams(
            dimension_semantics=("arbitrary", "arbitrary"),
        ),
    )(x, embedding)
    return d, a.reshape(_TSZ), loss.reshape(())


# speedup vs baseline: 2.3228x; 1.0020x over previous
import jax
import jax.numpy as jnp
from jax.experimental import pallas as pl
from jax.experimental.pallas import tpu as pltpu

_TSZ = 4096
_K = 8192
_D = 256
_GAMMA = 0.25

_BR = 1024
_BC = 2048
_NI = _TSZ // _BR
_NJ = _K // _BC


def _main_kernel(x_ref, e_ref, d_ref, a_ref, loss_ref,
                 zen_ref, en_ref, runm_ref, runa_ref, acc_ref):
    j = pl.program_id(0)
    i = pl.program_id(1)

    @pl.when(i == 0)
    def _():
        e = e_ref[:, 0, :]
        n1 = jnp.sqrt(jnp.sum(e * e, axis=1, keepdims=True))
        p1 = e / jnp.maximum(n1, 1e-12)
        n2 = jnp.sqrt(jnp.sum(p1 * p1, axis=1, keepdims=True))
        en_ref[...] = p1 / jnp.maximum(n2, 1e-8)

    @pl.when(jnp.logical_and(j == 0, i == 0))
    def _():
        ze = x_ref[0]
        an = jnp.sqrt(jnp.sum(ze * ze, axis=1, keepdims=True))
        zen_ref[...] = ze / jnp.maximum(an, 1e-8)

    dt = jax.lax.dot_general(
        zen_ref[pl.ds(i * _BR, _BR), :], en_ref[...],
        (((1,), (1,)), ((), ())),
        preferred_element_type=jnp.float32,
    )
    d_ref[...] = dt

    tile_max = jnp.max(dt, axis=1, keepdims=True)
    iota = jax.lax.broadcasted_iota(jnp.int32, dt.shape, 1)
    masked = jnp.where(dt == tile_max, iota, _K)
    tile_arg = jnp.min(masked, axis=1, keepdims=True) + j * _BC

    rows = pl.ds(i * _BR, _BR)

    @pl.when(j == 0)
    def _():
        runm_ref[rows, :] = tile_max
        runa_ref[rows, :] = tile_arg

    @pl.when(j > 0)
    def _():
        better = tile_max > runm_ref[rows, :]
        runa_ref[rows, :] = jnp.where(better, tile_arg, runa_ref[rows, :])
        runm_ref[rows, :] = jnp.maximum(tile_max, runm_ref[rows, :])

    @pl.when(j == _NJ - 1)
    def _():
        m = runm_ref[rows, :]
        a_ref[...] = runa_ref[rows, :]
        logp = jnp.maximum(jnp.log(jnp.clip(m, 1e-12, 1.0)), -100.0)
        part = jnp.sum(logp) * (-(1.0 + _GAMMA) / _TSZ)

        @pl.when(i == 0)
        def _():
            acc_ref[0] = part

        @pl.when(i > 0)
        def _():
            acc_ref[0] = acc_ref[0] + part

        @pl.when(i == _NI - 1)
        def _():
            loss_ref[...] = jnp.reshape(acc_ref[0], (1, 1))


def kernel(x, embedding):
    d, a, loss = pl.pallas_call(
        _main_kernel,
        grid=(_NJ, _NI),
        in_specs=[
            pl.BlockSpec((1, _TSZ, _D), lambda j, i: (0, 0, 0)),
            pl.BlockSpec((_BC, 1, _D), lambda j, i: (j, 0, 0)),
        ],
        out_specs=[
            pl.BlockSpec((_BR, _BC), lambda j, i: (i, j)),
            pl.BlockSpec((_BR, 1), lambda j, i: (i, 0)),
            pl.BlockSpec((1, 1), lambda j, i: (0, 0)),
        ],
        out_shape=[
            jax.ShapeDtypeStruct((_TSZ, _K), jnp.float32),
            jax.ShapeDtypeStruct((_TSZ, 1), jnp.int32),
            jax.ShapeDtypeStruct((1, 1), jnp.float32),
        ],
        scratch_shapes=[
            pltpu.VMEM((_TSZ, _D), jnp.float32),
            pltpu.VMEM((_BC, _D), jnp.float32),
            pltpu.VMEM((_TSZ, 1), jnp.float32),
            pltpu.VMEM((_TSZ, 1), jnp.int32),
            pltpu.SMEM((1,), jnp.float32),
        ],
        compiler_params=pltpu.CompilerParams(
            dimension_semantics=("arbitrary", "arbitrary"),
        ),
    )(x, embedding)
    return d, a.reshape(_TSZ), loss.reshape(())


# speedup vs baseline: 2.3537x; 1.0133x over previous
import jax
import jax.numpy as jnp
from jax.experimental import pallas as pl
from jax.experimental.pallas import tpu as pltpu

_TSZ = 4096
_K = 8192
_D = 256
_GAMMA = 0.25

_BR = 256
_NI = _TSZ // _BR


def _main_kernel(x_ref, e_ref, d_ref, a_ref, loss_ref, en_ref, acc_ref):
    i = pl.program_id(0)

    @pl.when(i == 0)
    def _():
        e = e_ref[:, 0, :]
        n1 = jnp.sqrt(jnp.sum(e * e, axis=1, keepdims=True))
        p1 = e / jnp.maximum(n1, 1e-12)
        n2 = jnp.sqrt(jnp.sum(p1 * p1, axis=1, keepdims=True))
        en_ref[...] = p1 / jnp.maximum(n2, 1e-8)

    ze = x_ref[0, pl.ds(i * _BR, _BR), :]
    an = jnp.sqrt(jnp.sum(ze * ze, axis=1, keepdims=True))
    zen = ze / jnp.maximum(an, 1e-8)

    dt = jax.lax.dot_general(
        zen, en_ref[...], (((1,), (1,)), ((), ())),
        preferred_element_type=jnp.float32,
    )
    d_ref[...] = dt

    m = jnp.max(dt, axis=1, keepdims=True)
    iota = jax.lax.broadcasted_iota(jnp.int32, dt.shape, 1)
    masked = jnp.where(dt == m, iota, _K)
    a_ref[...] = jnp.min(masked, axis=1, keepdims=True)

    logp = jnp.maximum(jnp.log(jnp.clip(m, 1e-12, 1.0)), -100.0)
    part = jnp.sum(logp) * (-(1.0 + _GAMMA) / _TSZ)

    @pl.when(i == 0)
    def _():
        acc_ref[0] = part

    @pl.when(i > 0)
    def _():
        acc_ref[0] = acc_ref[0] + part

    @pl.when(i == _NI - 1)
    def _():
        loss_ref[...] = jnp.reshape(acc_ref[0], (1, 1))


def kernel(x, embedding):
    d, a, loss = pl.pallas_call(
        _main_kernel,
        grid=(_NI,),
        in_specs=[
            pl.BlockSpec((1, _TSZ, _D), lambda i: (0, 0, 0)),
            pl.BlockSpec((_K, 1, _D), lambda i: (0, 0, 0)),
        ],
        out_specs=[
            pl.BlockSpec((_BR, _K), lambda i: (i, 0)),
            pl.BlockSpec((_BR, 1), lambda i: (i, 0)),
            pl.BlockSpec((1, 1), lambda i: (0, 0)),
        ],
        out_shape=[
            jax.ShapeDtypeStruct((_TSZ, _K), jnp.float32),
            jax.ShapeDtypeStruct((_TSZ, 1), jnp.int32),
            jax.ShapeDtypeStruct((1, 1), jnp.float32),
        ],
        scratch_shapes=[
            pltpu.VMEM((_K, _D), jnp.float32),
            pltpu.SMEM((1,), jnp.float32),
        ],
        compiler_params=pltpu.CompilerParams(
            dimension_semantics=("arbitrary",),
        ),
    )(x, embedding)
    return d, a.reshape(_TSZ), loss.reshape(())


# speedup vs baseline: 2.6452x; 1.1239x over previous
import jax
import jax.numpy as jnp
from jax.experimental import pallas as pl
from jax.experimental.pallas import tpu as pltpu

_TSZ = 4096
_K = 8192
_D = 256
_GAMMA = 0.25

_BR = 256
_NI = _TSZ // _BR


def _main_kernel(x_ref, e_ref, d_ref, a_ref, loss_ref, en_ref, acc_ref):
    i = pl.program_id(0)

    @pl.when(i == 0)
    def _():
        e = e_ref[:, 0, :]
        n1 = jnp.sqrt(jnp.sum(e * e, axis=1, keepdims=True))
        en_ref[...] = e / jnp.maximum(n1, 1e-12)

    ze = x_ref[0, pl.ds(i * _BR, _BR), :]
    an = jnp.sqrt(jnp.sum(ze * ze, axis=1, keepdims=True))
    zen = ze / jnp.maximum(an, 1e-8)

    dt = jax.lax.dot_general(
        zen, en_ref[...], (((1,), (1,)), ((), ())),
        preferred_element_type=jnp.float32,
    )
    d_ref[...] = dt

    m = jnp.max(dt, axis=1, keepdims=True)
    iota = jax.lax.broadcasted_iota(jnp.int32, dt.shape, 1)
    masked = jnp.where(dt == m, iota, _K)
    a_ref[...] = jnp.min(masked, axis=1, keepdims=True)

    logp = jnp.maximum(jnp.log(jnp.clip(m, 1e-12, 1.0)), -100.0)
    part = jnp.sum(logp) * (-(1.0 + _GAMMA) / _TSZ)

    @pl.when(i == 0)
    def _():
        acc_ref[0] = part

    @pl.when(i > 0)
    def _():
        acc_ref[0] = acc_ref[0] + part

    @pl.when(i == _NI - 1)
    def _():
        loss_ref[...] = jnp.reshape(acc_ref[0], (1, 1))


def kernel(x, embedding):
    d, a, loss = pl.pallas_call(
        _main_kernel,
        grid=(_NI,),
        in_specs=[
            pl.BlockSpec((1, _TSZ, _D), lambda i: (0, 0, 0)),
            pl.BlockSpec((_K, 1, _D), lambda i: (0, 0, 0)),
        ],
        out_specs=[
            pl.BlockSpec((_BR, _K), lambda i: (i, 0)),
            pl.BlockSpec((_BR, 1), lambda i: (i, 0)),
            pl.BlockSpec((1, 1), lambda i: (0, 0)),
        ],
        out_shape=[
            jax.ShapeDtypeStruct((_TSZ, _K), jnp.float32),
            jax.ShapeDtypeStruct((_TSZ, 1), jnp.int32),
            jax.ShapeDtypeStruct((1, 1), jnp.float32),
        ],
        scratch_shapes=[
            pltpu.VMEM((_K, _D), jnp.float32),
            pltpu.SMEM((1,), jnp.float32),
        ],
        compiler_params=pltpu.CompilerParams(
            dimension_semantics=("arbitrary",),
        ),
    )(x, embedding)
    return d, a.reshape(_TSZ), loss.reshape(())


# speedup vs baseline: 2.9136x; 1.1014x over previous
import jax
import jax.numpy as jnp
from jax.experimental import pallas as pl
from jax.experimental.pallas import tpu as pltpu

_TSZ = 4096
_K = 8192
_D = 256
_GAMMA = 0.25

_BR = 256
_NI = _TSZ // _BR


def _main_kernel(x_ref, e_ref, d_ref, a_ref, loss_ref, en_ref, acc_ref):
    i = pl.program_id(0)

    @pl.when(i == 0)
    def _():
        et = jnp.transpose(e_ref[:, 0, :])
        n1 = jnp.sqrt(jnp.sum(et * et, axis=0, keepdims=True))
        en_ref[...] = et / jnp.maximum(n1, 1e-12)

    ze = x_ref[0, pl.ds(i * _BR, _BR), :]
    an = jnp.sqrt(jnp.sum(ze * ze, axis=1, keepdims=True))
    zen = ze / jnp.maximum(an, 1e-8)

    dt = jax.lax.dot_general(
        zen, en_ref[...], (((1,), (0,)), ((), ())),
        preferred_element_type=jnp.float32,
    )
    d_ref[...] = dt

    m = jnp.max(dt, axis=1, keepdims=True)
    iota = jax.lax.broadcasted_iota(jnp.int32, dt.shape, 1)
    masked = jnp.where(dt == m, iota, _K)
    a_ref[...] = jnp.min(masked, axis=1, keepdims=True)

    logp = jnp.maximum(jnp.log(jnp.clip(m, 1e-12, 1.0)), -100.0)
    part = jnp.sum(logp) * (-(1.0 + _GAMMA) / _TSZ)

    @pl.when(i == 0)
    def _():
        acc_ref[0] = part

    @pl.when(i > 0)
    def _():
        acc_ref[0] = acc_ref[0] + part

    @pl.when(i == _NI - 1)
    def _():
        loss_ref[...] = jnp.reshape(acc_ref[0], (1, 1))


def kernel(x, embedding):
    d, a, loss = pl.pallas_call(
        _main_kernel,
        grid=(_NI,),
        in_specs=[
            pl.BlockSpec((1, _TSZ, _D), lambda i: (0, 0, 0)),
            pl.BlockSpec((_K, 1, _D), lambda i: (0, 0, 0)),
        ],
        out_specs=[
            pl.BlockSpec((_BR, _K), lambda i: (i, 0)),
            pl.BlockSpec((_BR, 1), lambda i: (i, 0)),
            pl.BlockSpec((1, 1), lambda i: (0, 0)),
        ],
        out_shape=[
            jax.ShapeDtypeStruct((_TSZ, _K), jnp.float32),
            jax.ShapeDtypeStruct((_TSZ, 1), jnp.int32),
            jax.ShapeDtypeStruct((1, 1), jnp.float32),
        ],
        scratch_shapes=[
            pltpu.VMEM((_D, _K), jnp.float32),
            pltpu.SMEM((1,), jnp.float32),
        ],
        compiler_params=pltpu.CompilerParams(
            dimension_semantics=("arbitrary",),
        ),
    )(x, embedding)
    return d, a.reshape(_TSZ), loss.reshape(())
